# Initial kernel scaffold; baseline (speedup 1.0000x reference)
import jax
import jax.numpy as jnp
from jax.experimental import pallas as pl


def kernel(x, table):
    raise NotImplementedError("write your pallas kernel here")


# speedup vs baseline: 6.1294x; 6.1294x over previous
import functools
import math

import jax
import jax.numpy as jnp
from jax import lax
from jax.experimental import pallas as pl
from jax.experimental.pallas import tpu as pltpu
from jax.experimental.pallas import tpu_sc as plsc

_NC = 2
_NS = 16
_NW = _NC * _NS

_CHUNK = 256


def _scale_body(scale_val, t_ref, o_ref):
    o_ref[...] = t_ref[...] * scale_val


def _scale_table(table):
    v, d = table.shape
    blk = 2000
    assert v % blk == 0
    scale = math.sqrt(float(d))
    return pl.pallas_call(
        functools.partial(_scale_body, scale),
        out_shape=jax.ShapeDtypeStruct((v, d), table.dtype),
        grid=(v // blk,),
        in_specs=[pl.BlockSpec((blk, d), lambda i: (i, 0))],
        out_specs=pl.BlockSpec((blk, d), lambda i: (i, 0)),
    )(table)


@functools.lru_cache(maxsize=None)
def _make_gather(B, D):
    assert B % (8 * _NW) == 0
    b_per_w = B // _NW
    assert b_per_w % _CHUNK == 0
    steps = b_per_w // _CHUNK
    mesh = plsc.VectorSubcoreMesh(core_axis_name="c", subcore_axis_name="s")

    @functools.partial(
        pl.kernel,
        mesh=mesh,
        out_type=jax.ShapeDtypeStruct((B, D), jnp.float32),
        scratch_types=[
            pltpu.VMEM((_CHUNK,), jnp.int32),
            pltpu.VMEM((_CHUNK, D), jnp.float32),
            pltpu.SemaphoreType.DMA,
        ],
    )
    def gather_kernel(table_hbm, idx_hbm, out_hbm, idx_v, rows_v, sem):
        wid = lax.axis_index("s") * _NC + lax.axis_index("c")
        base = wid * b_per_w

        def body(j, carry):
            off = base + j * _CHUNK
            pltpu.sync_copy(idx_hbm.at[pl.ds(off, _CHUNK)], idx_v)
            pltpu.async_copy(table_hbm.at[idx_v], rows_v, sem).wait()
            pltpu.sync_copy(rows_v, out_hbm.at[pl.ds(off, _CHUNK)])
            return carry

        lax.fori_loop(0, steps, body, 0)

    return gather_kernel


def kernel(x, table):
    n, s = x.shape
    v, d = table.shape
    scaled = _scale_table(table)
    idx = x.reshape(n * s).astype(jnp.int32)
    out = _make_gather(n * s, d)(scaled, idx)
    return out.reshape(n, s, d)


# speedup vs baseline: 7.9412x; 1.2956x over previous
import functools
import math

import jax
import jax.numpy as jnp
from jax import lax
from jax.experimental import pallas as pl
from jax.experimental.pallas import tpu as pltpu
from jax.experimental.pallas import tpu_sc as plsc

_NC = 2
_NS = 16
_NW = _NC * _NS

_CHUNK = 256


def _scale_body(scale_val, t_ref, o_ref):
    o_ref[...] = t_ref[...] * scale_val


def _scale_table(table):
    v, d = table.shape
    blk = 2000
    assert v % blk == 0
    scale = math.sqrt(float(d))
    return pl.pallas_call(
        functools.partial(_scale_body, scale),
        out_shape=jax.ShapeDtypeStruct((v, d), table.dtype),
        grid=(v // blk,),
        in_specs=[pl.BlockSpec((blk, d), lambda i: (i, 0))],
        out_specs=pl.BlockSpec((blk, d), lambda i: (i, 0)),
    )(table)


@functools.lru_cache(maxsize=None)
def _make_gather(B, D):
    assert B % (8 * _NW) == 0
    b_per_w = B // _NW
    assert b_per_w % _CHUNK == 0
    steps = b_per_w // _CHUNK
    assert steps % 2 == 0 and steps >= 4
    mesh = plsc.VectorSubcoreMesh(core_axis_name="c", subcore_axis_name="s")

    @functools.partial(
        pl.kernel,
        mesh=mesh,
        out_type=jax.ShapeDtypeStruct((B, D), jnp.float32),
        scratch_types=[
            pltpu.VMEM((b_per_w,), jnp.int32),
            pltpu.VMEM((_CHUNK, D), jnp.float32),
            pltpu.VMEM((_CHUNK, D), jnp.float32),
            pltpu.SemaphoreType.DMA,
            pltpu.SemaphoreType.DMA,
        ],
    )
    def gather_kernel(table_hbm, idx_hbm, out_hbm, idx_v, rows0, rows1, g0, g1):
        wid = lax.axis_index("s") * _NC + lax.axis_index("c")
        base = wid * b_per_w

        def start_gather(j, rows, sem):
            pltpu.async_copy(
                table_hbm.at[idx_v.at[pl.ds(j * _CHUNK, _CHUNK)]], rows, sem)

        def wait_gather(rows, sem):
            pltpu.make_async_copy(
                table_hbm.at[idx_v.at[pl.ds(0, _CHUNK)]], rows, sem).wait()

        def store(j, rows):
            pltpu.sync_copy(rows, out_hbm.at[pl.ds(base + j * _CHUNK, _CHUNK)])

        pltpu.sync_copy(idx_hbm.at[pl.ds(base, b_per_w)], idx_v)
        start_gather(0, rows0, g0)

        def body(g, carry):
            j0 = 2 * g
            start_gather(j0 + 1, rows1, g1)
            wait_gather(rows0, g0)
            store(j0, rows0)
            start_gather(j0 + 2, rows0, g0)
            wait_gather(rows1, g1)
            store(j0 + 1, rows1)
            return carry

        lax.fori_loop(0, steps // 2 - 1, body, 0)

        start_gather(steps - 1, rows1, g1)
        wait_gather(rows0, g0)
        store(steps - 2, rows0)
        wait_gather(rows1, g1)
        store(steps - 1, rows1)

    return gather_kernel


def kernel(x, table):
    n, s = x.shape
    v, d = table.shape
    scaled = _scale_table(table)
    b = n * s
    idx = x.reshape(b).astype(jnp.int32)
    out = _make_gather(b, d)(scaled, idx)
    return out.reshape(n, s, d)


# speedup vs baseline: 7.9604x; 1.0024x over previous
import functools
import math

import jax
import jax.numpy as jnp
from jax import lax
from jax.experimental import pallas as pl
from jax.experimental.pallas import tpu as pltpu
from jax.experimental.pallas import tpu_sc as plsc

_NC = 2
_NS = 16
_NW = _NC * _NS

_CHUNK = 160
_NBUF = 4


def _scale_body(scale_val, t_ref, o_ref):
    o_ref[...] = t_ref[...] * scale_val


def _scale_table(table):
    v, d = table.shape
    blk = 2000
    assert v % blk == 0
    scale = math.sqrt(float(d))
    return pl.pallas_call(
        functools.partial(_scale_body, scale),
        out_shape=jax.ShapeDtypeStruct((v, d), table.dtype),
        grid=(v // blk,),
        in_specs=[pl.BlockSpec((blk, d), lambda i: (i, 0))],
        out_specs=pl.BlockSpec((blk, d), lambda i: (i, 0)),
    )(table)


@functools.lru_cache(maxsize=None)
def _make_gather(B, D):
    assert B % (8 * _NW) == 0
    b_per_w = B // _NW
    assert b_per_w % _CHUNK == 0
    steps = b_per_w // _CHUNK
    assert steps % _NBUF == 0 and steps >= 3 * _NBUF
    mesh = plsc.VectorSubcoreMesh(core_axis_name="c", subcore_axis_name="s")

    @functools.partial(
        pl.kernel,
        mesh=mesh,
        out_type=jax.ShapeDtypeStruct((B, D), jnp.float32),
        scratch_types=[
            pltpu.VMEM((b_per_w,), jnp.int32),
        ] + [pltpu.VMEM((_CHUNK, D), jnp.float32)] * _NBUF
          + [pltpu.SemaphoreType.DMA] * (2 * _NBUF),
    )
    def gather_kernel(table_hbm, idx_hbm, out_hbm, idx_v, *bufs_and_sems):
        rows = bufs_and_sems[:_NBUF]
        gs = bufs_and_sems[_NBUF:2 * _NBUF]
        sts = bufs_and_sems[2 * _NBUF:]
        wid = lax.axis_index("s") * _NC + lax.axis_index("c")
        base = wid * b_per_w

        def start_gather(j, u):
            pltpu.async_copy(
                table_hbm.at[idx_v.at[pl.ds(j * _CHUNK, _CHUNK)]], rows[u], gs[u])

        def wait_gather(u):
            pltpu.make_async_copy(
                table_hbm.at[idx_v.at[pl.ds(0, _CHUNK)]], rows[u], gs[u]).wait()

        def start_store(j, u):
            pltpu.async_copy(
                rows[u], out_hbm.at[pl.ds(base + j * _CHUNK, _CHUNK)], sts[u])

        def wait_store(u):
            pltpu.make_async_copy(
                rows[u], out_hbm.at[pl.ds(base, _CHUNK)], sts[u]).wait()

        def full(j, u, st_wait=True):
            bg = (u + 2) % _NBUF
            if st_wait:
                wait_store(bg)
            start_gather(j + 2, bg)
            wait_gather(u)
            start_store(j, u)

        def tail(j, u):
            wait_gather(u)
            start_store(j, u)

        pltpu.sync_copy(idx_hbm.at[pl.ds(base, b_per_w)], idx_v)
        start_gather(0, 0)
        start_gather(1, 1)
        full(0, 0, st_wait=False)
        full(1, 1, st_wait=False)
        full(2, 2)
        full(3, 3)

        def body(gp, carry):
            j0 = _NBUF * gp + _NBUF
            for u in range(_NBUF):
                full(j0 + u, u)
            return carry

        lax.fori_loop(0, (steps - 2 * _NBUF) // _NBUF, body, 0)

        full(steps - 4, 0)
        full(steps - 3, 1)
        tail(steps - 2, 2)
        tail(steps - 1, 3)
        for u in range(_NBUF):
            wait_store(u)

    return gather_kernel


def kernel(x, table):
    n, s = x.shape
    v, d = table.shape
    scaled = _scale_table(table)
    b = n * s
    idx = x.reshape(b).astype(jnp.int32)
    out = _make_gather(b, d)(scaled, idx)
    return out.reshape(n, s, d)


# speedup vs baseline: 9.1177x; 1.1454x over previous
import functools
import math

import jax
import jax.numpy as jnp
from jax import lax
from jax.experimental import pallas as pl
from jax.experimental.pallas import tpu as pltpu
from jax.experimental.pallas import tpu_sc as plsc

_NC = 2
_NS = 16
_NW = _NC * _NS

_CHUNK = 160
_NBUF = 4
_L = 16
_RPI = 4


@functools.lru_cache(maxsize=None)
def _make_gather(B, D):
    assert B % (8 * _NW) == 0
    b_per_w = B // _NW
    assert b_per_w % _CHUNK == 0
    steps = b_per_w // _CHUNK
    assert steps % _NBUF == 0 and steps >= 3 * _NBUF
    assert _CHUNK % _RPI == 0 and D % _L == 0
    scale = math.sqrt(float(D))
    mesh = plsc.VectorSubcoreMesh(core_axis_name="c", subcore_axis_name="s")

    @functools.partial(
        pl.kernel,
        mesh=mesh,
        out_type=jax.ShapeDtypeStruct((B, D), jnp.float32),
        scratch_types=[
            pltpu.VMEM((b_per_w,), jnp.int32),
        ] + [pltpu.VMEM((_CHUNK, D), jnp.float32)] * _NBUF
          + [pltpu.SemaphoreType.DMA] * (2 * _NBUF),
    )
    def gather_kernel(table_hbm, idx_hbm, out_hbm, idx_v, *bufs_and_sems):
        rows = bufs_and_sems[:_NBUF]
        gs = bufs_and_sems[_NBUF:2 * _NBUF]
        sts = bufs_and_sems[2 * _NBUF:]
        wid = lax.axis_index("s") * _NC + lax.axis_index("c")
        base = wid * b_per_w

        def start_gather(j, u):
            pltpu.async_copy(
                table_hbm.at[idx_v.at[pl.ds(j * _CHUNK, _CHUNK)]], rows[u], gs[u])

        def wait_gather(u):
            pltpu.make_async_copy(
                table_hbm.at[idx_v.at[pl.ds(0, _CHUNK)]], rows[u], gs[u]).wait()

        def start_store(j, u):
            pltpu.async_copy(
                rows[u], out_hbm.at[pl.ds(base + j * _CHUNK, _CHUNK)], sts[u])

        def wait_store(u):
            pltpu.make_async_copy(
                rows[u], out_hbm.at[pl.ds(base, _CHUNK)], sts[u]).wait()

        def scale_buf(u):
            r = rows[u]

            def srow(i, carry):
                for rr in range(_RPI):
                    for c in range(D // _L):
                        sl = pl.ds(c * _L, _L)
                        r[i * _RPI + rr, sl] = r[i * _RPI + rr, sl] * scale
                return carry

            lax.fori_loop(0, _CHUNK // _RPI, srow, 0)

        def full(j, u, st_wait=True):
            bg = (u + 2) % _NBUF
            if st_wait:
                wait_store(bg)
            start_gather(j + 2, bg)
            wait_gather(u)
            scale_buf(u)
            start_store(j, u)

        def tail(j, u):
            wait_gather(u)
            scale_buf(u)
            start_store(j, u)

        pltpu.sync_copy(idx_hbm.at[pl.ds(base, b_per_w)], idx_v)
        start_gather(0, 0)
        start_gather(1, 1)
        full(0, 0, st_wait=False)
        full(1, 1, st_wait=False)
        full(2, 2)
        full(3, 3)

        def body(gp, carry):
            j0 = _NBUF * gp + _NBUF
            for u in range(_NBUF):
                full(j0 + u, u)
            return carry

        lax.fori_loop(0, (steps - 2 * _NBUF) // _NBUF, body, 0)

        full(steps - 4, 0)
        full(steps - 3, 1)
        tail(steps - 2, 2)
        tail(steps - 1, 3)
        for u in range(_NBUF):
            wait_store(u)

    return gather_kernel


def kernel(x, table):
    n, s = x.shape
    v, d = table.shape
    b = n * s
    idx = x.reshape(b).astype(jnp.int32)
    out = _make_gather(b, d)(table, idx)
    return out.reshape(n, s, d)


# speedup vs baseline: 9.1676x; 1.0055x over previous
import functools
import math

import jax
import jax.numpy as jnp
from jax import lax
from jax.experimental import pallas as pl
from jax.experimental.pallas import tpu as pltpu
from jax.experimental.pallas import tpu_sc as plsc

_NC = 2
_NS = 16
_NW = _NC * _NS

_CHUNK = 200
_NBUF = 4
_L = 16
_RPI = 4


@functools.lru_cache(maxsize=None)
def _make_gather(B, D):
    assert B % (8 * _NW) == 0
    b_per_w = B // _NW
    assert b_per_w % _CHUNK == 0
    steps = b_per_w // _CHUNK
    assert steps % _NBUF == 0 and steps >= 3 * _NBUF
    assert _CHUNK % _RPI == 0 and D % _L == 0
    scale = math.sqrt(float(D))
    mesh = plsc.VectorSubcoreMesh(core_axis_name="c", subcore_axis_name="s")

    @functools.partial(
        pl.kernel,
        mesh=mesh,
        out_type=jax.ShapeDtypeStruct((B, D), jnp.float32),
        scratch_types=[
            pltpu.VMEM((b_per_w,), jnp.int32),
        ] + [pltpu.VMEM((_CHUNK, D), jnp.float32)] * _NBUF
          + [pltpu.SemaphoreType.DMA] * (2 * _NBUF),
    )
    def gather_kernel(table_hbm, idx_hbm, out_hbm, idx_v, *bufs_and_sems):
        rows = bufs_and_sems[:_NBUF]
        gs = bufs_and_sems[_NBUF:2 * _NBUF]
        sts = bufs_and_sems[2 * _NBUF:]
        wid = lax.axis_index("s") * _NC + lax.axis_index("c")
        base = wid * b_per_w

        def start_gather(j, u):
            pltpu.async_copy(
                table_hbm.at[idx_v.at[pl.ds(j * _CHUNK, _CHUNK)]], rows[u], gs[u])

        def wait_gather(u):
            pltpu.make_async_copy(
                table_hbm.at[idx_v.at[pl.ds(0, _CHUNK)]], rows[u], gs[u]).wait()

        def start_store(j, u):
            pltpu.async_copy(
                rows[u], out_hbm.at[pl.ds(base + j * _CHUNK, _CHUNK)], sts[u])

        def wait_store(u):
            pltpu.make_async_copy(
                rows[u], out_hbm.at[pl.ds(base, _CHUNK)], sts[u]).wait()

        def scale_buf(u):
            r = rows[u]

            def srow(i, carry):
                for rr in range(_RPI):
                    for c in range(D // _L):
                        sl = pl.ds(c * _L, _L)
                        r[i * _RPI + rr, sl] = r[i * _RPI + rr, sl] * scale
                return carry

            lax.fori_loop(0, _CHUNK // _RPI, srow, 0)

        def full(j, u, st_wait=True):
            bg = (u + 2) % _NBUF
            if st_wait:
                wait_store(bg)
            start_gather(j + 2, bg)
            wait_gather(u)
            scale_buf(u)
            start_store(j, u)

        def tail(j, u):
            wait_gather(u)
            scale_buf(u)
            start_store(j, u)

        pltpu.sync_copy(idx_hbm.at[pl.ds(base, b_per_w)], idx_v)
        start_gather(0, 0)
        start_gather(1, 1)
        full(0, 0, st_wait=False)
        full(1, 1, st_wait=False)
        full(2, 2)
        full(3, 3)

        def body(gp, carry):
            j0 = _NBUF * gp + _NBUF
            for u in range(_NBUF):
                full(j0 + u, u)
            return carry

        lax.fori_loop(0, (steps - 2 * _NBUF) // _NBUF, body, 0)

        full(steps - 4, 0)
        full(steps - 3, 1)
        tail(steps - 2, 2)
        tail(steps - 1, 3)
        for u in range(_NBUF):
            wait_store(u)

    return gather_kernel


def kernel(x, table):
    n, s = x.shape
    v, d = table.shape
    b = n * s
    idx = x.reshape(b).astype(jnp.int32)
    out = _make_gather(b, d)(table, idx)
    return out.reshape(n, s, d)
